# Initial kernel scaffold; baseline (speedup 1.0000x reference)
import jax
import jax.numpy as jnp
from jax import lax
from jax.experimental import pallas as pl
from jax.experimental.pallas import tpu as pltpu
from jax.experimental.pallas import tpu_sc as plsc

K = 8192
D = 64
B = 64
NPIX = 1024
L = 16
NC = 2
NS = 16
IMGS_PER_W = B // (NC * NS)


def _table_body(cb_ref, w_ref, b_ref, out_ref):
    logits = jnp.dot(cb_ref[...], w_ref[...],
                     preferred_element_type=jnp.float32)
    out_ref[...] = jax.nn.sigmoid(logits + b_ref[...])


def _gather_body(tbl_e_hbm, tbl_n_hbm, z_hbm,
                 out_e_hbm, out_n_hbm, alea_hbm, epis_hbm,
                 tbl_e_v, tbl_n_v, idx_v, out_e_v, out_n_v, mrow_v):
    wid = lax.axis_index("s") * NC + lax.axis_index("c")
    pltpu.sync_copy(tbl_e_hbm, tbl_e_v)
    pltpu.sync_copy(tbl_n_hbm, tbl_n_v)
    for t in range(IMGS_PER_W):
        img = wid * IMGS_PER_W + t
        pltpu.sync_copy(z_hbm.at[pl.ds(img * NPIX, NPIX)], idx_v)

        def body(j, accs):
            acc_e, acc_n = accs
            idx = idx_v[pl.ds(j * L, L)]
            e = plsc.load_gather(tbl_e_v, [idx])
            n = plsc.load_gather(tbl_n_v, [idx])
            out_e_v[pl.ds(j * L, L)] = e
            out_n_v[pl.ds(j * L, L)] = n
            return (acc_e + e, acc_n + n)

        zero = jnp.zeros((L,), jnp.float32)
        acc_e, acc_n = lax.fori_loop(0, NPIX // L, body, (zero, zero))
        pltpu.sync_copy(out_e_v, out_e_hbm.at[img])
        pltpu.sync_copy(out_n_v, out_n_hbm.at[img])
        mrow_v[...] = jnp.full((L,), jnp.sum(acc_e) * (1.0 / NPIX),
                               jnp.float32)
        pltpu.sync_copy(mrow_v, alea_hbm.at[img])
        mrow_v[...] = jnp.full((L,), jnp.sum(acc_n) * (1.0 / NPIX),
                               jnp.float32)
        pltpu.sync_copy(mrow_v, epis_hbm.at[img])


def kernel(z, codebook, lin_w, lin_b):
    table = pl.pallas_call(
        _table_body,
        out_shape=jax.ShapeDtypeStruct((K, 2), jnp.float32),
    )(codebook, lin_w, lin_b.reshape(1, 2))
    tbl_e = table[:, 0]
    tbl_n = table[:, 1]
    zf = z.reshape(-1).astype(jnp.int32)

    mesh = plsc.VectorSubcoreMesh(core_axis_name="c", subcore_axis_name="s")
    sc = pl.kernel(
        _gather_body,
        mesh=mesh,
        out_type=[
            jax.ShapeDtypeStruct((B, NPIX), jnp.float32),
            jax.ShapeDtypeStruct((B, NPIX), jnp.float32),
            jax.ShapeDtypeStruct((B, L), jnp.float32),
            jax.ShapeDtypeStruct((B, L), jnp.float32),
        ],
        scratch_types=[
            pltpu.VMEM((K,), jnp.float32),
            pltpu.VMEM((K,), jnp.float32),
            pltpu.VMEM((NPIX,), jnp.int32),
            pltpu.VMEM((NPIX,), jnp.float32),
            pltpu.VMEM((NPIX,), jnp.float32),
            pltpu.VMEM((L,), jnp.float32),
        ],
    )
    out_e, out_n, alea_b, epis_b = sc(tbl_e, tbl_n, zf)
    endosome = out_e.reshape(B, 1, 32, 32)
    nuclear = out_n.reshape(B, 1, 32, 32)
    alea = alea_b[:, :1]
    epis = epis_b[:, :1]
    return (endosome, nuclear, alea, epis)


# speedup vs baseline: 4.9206x; 4.9206x over previous
import jax
import jax.numpy as jnp
from jax import lax
from jax.experimental import pallas as pl
from jax.experimental.pallas import tpu as pltpu
from jax.experimental.pallas import tpu_sc as plsc

K = 8192
D = 64
B = 64
NPIX = 1024
L = 16
NC = 2
NS = 16
IMGS_PER_W = B // (NC * NS)


def _table_body(cb_ref, w_ref, b_ref, out_ref):
    logits = jnp.dot(cb_ref[...], w_ref[...],
                     preferred_element_type=jnp.float32)
    out_ref[...] = jax.nn.sigmoid(logits + b_ref[...])


def _gather_body(tbl_e_hbm, tbl_n_hbm, z_hbm,
                 out_e_hbm, out_n_hbm, alea_hbm, epis_hbm,
                 tbl_e_v, tbl_n_v, idx_v, out_e_v, out_n_v, mrow_v):
    wid = lax.axis_index("s") * NC + lax.axis_index("c")
    pltpu.sync_copy(tbl_e_hbm, tbl_e_v)
    pltpu.sync_copy(tbl_n_hbm, tbl_n_v)
    for t in range(IMGS_PER_W):
        img = wid * IMGS_PER_W + t
        pltpu.sync_copy(z_hbm.at[pl.ds(img * NPIX, NPIX)], idx_v)

        def body(j, accs):
            acc_e, acc_n = accs
            idx = idx_v[pl.ds(j * L, L)]
            e = plsc.load_gather(tbl_e_v, [idx])
            n = plsc.load_gather(tbl_n_v, [idx])
            out_e_v[pl.ds(j * L, L)] = e
            out_n_v[pl.ds(j * L, L)] = n
            return (acc_e + e, acc_n + n)

        zero = jnp.zeros((L,), jnp.float32)
        acc_e, acc_n = lax.fori_loop(0, NPIX // L, body, (zero, zero))
        pltpu.sync_copy(out_e_v, out_e_hbm.at[img])
        pltpu.sync_copy(out_n_v, out_n_hbm.at[img])
        mrow_v[...] = jnp.full((L,), jnp.sum(acc_e) * (1.0 / NPIX),
                               jnp.float32)
        pltpu.sync_copy(mrow_v, alea_hbm.at[img])
        mrow_v[...] = jnp.full((L,), jnp.sum(acc_n) * (1.0 / NPIX),
                               jnp.float32)
        pltpu.sync_copy(mrow_v, epis_hbm.at[img])


def kernel(z, codebook, lin_w, lin_b):
    table = pl.pallas_call(
        _table_body,
        out_shape=jax.ShapeDtypeStruct((K, 2), jnp.float32),
    )(codebook, lin_w, lin_b.reshape(1, 2))
    tbl_e = table[:, 0]
    tbl_n = table[:, 1]
    zf = z.reshape(-1).astype(jnp.int32)

    mesh = plsc.VectorSubcoreMesh(core_axis_name="c", subcore_axis_name="s")
    sc = pl.kernel(
        _gather_body,
        mesh=mesh,
        compiler_params=pltpu.CompilerParams(needs_layout_passes=False),
        out_type=[
            jax.ShapeDtypeStruct((B, NPIX), jnp.float32),
            jax.ShapeDtypeStruct((B, NPIX), jnp.float32),
            jax.ShapeDtypeStruct((B, L), jnp.float32),
            jax.ShapeDtypeStruct((B, L), jnp.float32),
        ],
        scratch_types=[
            pltpu.VMEM((K,), jnp.float32),
            pltpu.VMEM((K,), jnp.float32),
            pltpu.VMEM((NPIX,), jnp.int32),
            pltpu.VMEM((NPIX,), jnp.float32),
            pltpu.VMEM((NPIX,), jnp.float32),
            pltpu.VMEM((L,), jnp.float32),
        ],
    )
    out_e, out_n, alea_b, epis_b = sc(tbl_e, tbl_n, zf)
    endosome = out_e.reshape(B, 1, 32, 32)
    nuclear = out_n.reshape(B, 1, 32, 32)
    alea = alea_b[:, :1]
    epis = epis_b[:, :1]
    return (endosome, nuclear, alea, epis)


# speedup vs baseline: 5.2553x; 1.0680x over previous
import jax
import jax.numpy as jnp
from jax import lax
from jax.experimental import pallas as pl
from jax.experimental.pallas import tpu as pltpu
from jax.experimental.pallas import tpu_sc as plsc

K = 8192
D = 64
B = 64
NPIX = 1024
L = 16
NC = 2
NS = 16
IMGS_PER_W = B // (NC * NS)


def _table_body(cb_ref, w_ref, b_ref, out_ref):
    logits = jnp.dot(cb_ref[...], w_ref[...],
                     preferred_element_type=jnp.float32)
    out_ref[...] = jax.nn.sigmoid(logits + b_ref[...])


def _gather_body(tbl_hbm, z_hbm,
                 out_e_hbm, out_n_hbm, alea_hbm, epis_hbm,
                 tbl_v, idx_v, out_e_v, out_n_v, mrow_v):
    wid = lax.axis_index("s") * NC + lax.axis_index("c")
    pltpu.sync_copy(tbl_hbm, tbl_v)
    for t in range(IMGS_PER_W):
        img = wid * IMGS_PER_W + t
        pltpu.sync_copy(z_hbm.at[pl.ds(img * NPIX, NPIX)], idx_v)

        def body(j, accs):
            acc_e, acc_n = accs
            idx = idx_v[pl.ds(j * L, L)]
            idx2 = idx + idx
            e = plsc.load_gather(tbl_v, [idx2])
            n = plsc.load_gather(tbl_v, [idx2 + 1])
            out_e_v[pl.ds(j * L, L)] = e
            out_n_v[pl.ds(j * L, L)] = n
            return (acc_e + e, acc_n + n)

        zero = jnp.zeros((L,), jnp.float32)
        acc_e, acc_n = lax.fori_loop(0, NPIX // L, body, (zero, zero))
        pltpu.sync_copy(out_e_v, out_e_hbm.at[img])
        pltpu.sync_copy(out_n_v, out_n_hbm.at[img])
        mrow_v[...] = jnp.full((L,), jnp.sum(acc_e) * (1.0 / NPIX),
                               jnp.float32)
        pltpu.sync_copy(mrow_v, alea_hbm.at[img])
        mrow_v[...] = jnp.full((L,), jnp.sum(acc_n) * (1.0 / NPIX),
                               jnp.float32)
        pltpu.sync_copy(mrow_v, epis_hbm.at[img])


def kernel(z, codebook, lin_w, lin_b):
    table = pl.pallas_call(
        _table_body,
        out_shape=jax.ShapeDtypeStruct((K, 2), jnp.float32),
    )(codebook, lin_w, lin_b.reshape(1, 2))
    tbl = table.reshape(-1)
    zf = z.reshape(-1).astype(jnp.int32)

    mesh = plsc.VectorSubcoreMesh(core_axis_name="c", subcore_axis_name="s")
    sc = pl.kernel(
        _gather_body,
        mesh=mesh,
        compiler_params=pltpu.CompilerParams(needs_layout_passes=False),
        out_type=[
            jax.ShapeDtypeStruct((B, NPIX), jnp.float32),
            jax.ShapeDtypeStruct((B, NPIX), jnp.float32),
            jax.ShapeDtypeStruct((B, L), jnp.float32),
            jax.ShapeDtypeStruct((B, L), jnp.float32),
        ],
        scratch_types=[
            pltpu.VMEM((2 * K,), jnp.float32),
            pltpu.VMEM((NPIX,), jnp.int32),
            pltpu.VMEM((NPIX,), jnp.float32),
            pltpu.VMEM((NPIX,), jnp.float32),
            pltpu.VMEM((L,), jnp.float32),
        ],
    )
    out_e, out_n, alea_b, epis_b = sc(tbl, zf)
    endosome = out_e.reshape(B, 1, 32, 32)
    nuclear = out_n.reshape(B, 1, 32, 32)
    alea = alea_b[:, :1]
    epis = epis_b[:, :1]
    return (endosome, nuclear, alea, epis)
